# Initial kernel scaffold; baseline (speedup 1.0000x reference)
import functools

import jax
import jax.numpy as jnp
from jax import lax
from jax.experimental import pallas as pl
from jax.experimental.pallas import tpu as pltpu
from jax.experimental.pallas import tpu_sc as plsc


def _matmul_body(x_ref, w_ref, o_ref):
    o_ref[...] = jnp.dot(x_ref[...], w_ref[...],
                         preferred_element_type=jnp.float32)


def _matmul(x, W, row_block=1000):
    n, d = x.shape
    c = W.shape[1]
    grid = (n // row_block,)
    return pl.pallas_call(
        _matmul_body,
        grid=grid,
        in_specs=[
            pl.BlockSpec((row_block, d), lambda i: (i, 0)),
            pl.BlockSpec((d, c), lambda i: (0, 0)),
        ],
        out_specs=pl.BlockSpec((row_block, c), lambda i: (i, 0)),
        out_shape=jax.ShapeDtypeStruct((n, c), jnp.float32),
    )(x, W)


_BLK = 128


def _make_sc_agg(n_nodes, e_pad, n_classes):
    info = plsc.get_sparse_core_info()
    nc, ns = info.num_cores, info.num_subcores
    nw = nc * ns
    epw = e_pad // nw
    n_blk = epw // _BLK
    rows_per_tile = n_nodes // ns

    mesh = plsc.VectorSubcoreMesh(core_axis_name="c", subcore_axis_name="s")

    @functools.partial(
        pl.kernel,
        mesh=mesh,
        out_type=jax.ShapeDtypeStruct((nc, n_nodes, n_classes), jnp.float32),
        scratch_types=[
            pltpu.VMEM((_BLK,), jnp.int32),
            pltpu.VMEM((_BLK,), jnp.int32),
            pltpu.VMEM((_BLK,), jnp.float32),
            pltpu.VMEM((_BLK, n_classes), jnp.float32),
            pltpu.VMEM((n_nodes // ns, n_classes), jnp.float32),
            pltpu.VMEM_SHARED((n_nodes, n_classes), jnp.float32),
            pltpu.SemaphoreType.DMA,
        ],
    )
    def sc_agg(src_hbm, dst_hbm, w_hbm, h_hbm, out_hbm,
               sidx, didx, wv, rows, zbuf, acc, gsem):
        cid = lax.axis_index("c")
        sid = lax.axis_index("s")
        wid = sid * nc + cid
        my_out_base = sid * rows_per_tile

        def _zero(i, _):
            zbuf[i, :] = jnp.zeros((n_classes,), jnp.float32)
            return 0
        lax.fori_loop(0, rows_per_tile, _zero, 0)
        pltpu.sync_copy(zbuf, acc.at[pl.ds(my_out_base, rows_per_tile)])
        plsc.subcore_barrier()

        def _block(i, _):
            base = wid * epw + i * _BLK
            pltpu.sync_copy(src_hbm.at[pl.ds(base, _BLK)], sidx)
            pltpu.sync_copy(w_hbm.at[pl.ds(base, _BLK)], wv)
            pltpu.async_copy(h_hbm.at[sidx], rows, gsem).wait()
            for j0 in range(0, _BLK, 16):
                w16 = wv[pl.ds(j0, 16)]
                for j in range(16):
                    rows[j0 + j, :] = rows[j0 + j, :] * jnp.broadcast_to(
                        w16[j], (n_classes,))
            pltpu.sync_copy(dst_hbm.at[pl.ds(base, _BLK)], didx)
            pltpu.sync_copy(rows, acc.at[didx], add=True)
            return 0
        lax.fori_loop(0, n_blk, _block, 0)
        plsc.subcore_barrier()

        pltpu.sync_copy(acc.at[pl.ds(my_out_base, rows_per_tile)],
                        out_hbm.at[cid, pl.ds(my_out_base, rows_per_tile)])

    return sc_agg


def _lsm_body(p_ref, b_ref, o_ref):
    s = p_ref[0] + p_ref[1] + b_ref[...]
    m = jnp.max(s, axis=1, keepdims=True)
    e = jnp.exp(s - m)
    lse = jnp.log(jnp.sum(e, axis=1, keepdims=True))
    o_ref[...] = s - m - lse


def _log_softmax(parts, b, row_block=1000):
    _, n, c = parts.shape
    grid = (n // row_block,)
    return pl.pallas_call(
        _lsm_body,
        grid=grid,
        in_specs=[
            pl.BlockSpec((2, row_block, c), lambda i: (0, i, 0)),
            pl.BlockSpec((c,), lambda i: (0,)),
        ],
        out_specs=pl.BlockSpec((row_block, c), lambda i: (i, 0)),
        out_shape=jax.ShapeDtypeStruct((n, c), jnp.float32),
    )(parts, b)


@jax.jit
def kernel(x, edge_index, edge_weight, W, b):
    n_nodes = x.shape[0]
    n_edges = edge_index.shape[1]
    n_classes = W.shape[1]

    info = plsc.get_sparse_core_info()
    nw = info.num_cores * info.num_subcores
    chunk = nw * _BLK
    e_pad = ((n_edges + chunk - 1) // chunk) * chunk

    src = jnp.pad(edge_index[0], (0, e_pad - n_edges))
    dst = jnp.pad(edge_index[1], (0, e_pad - n_edges))
    w = jnp.pad(edge_weight, (0, e_pad - n_edges))

    h = _matmul(x, W)
    parts = _make_sc_agg(n_nodes, e_pad, n_classes)(src, dst, w, h)
    return _log_softmax(parts, b)


# speedup vs baseline: 6.9089x; 6.9089x over previous
import functools

import jax
import jax.numpy as jnp
from jax import lax
from jax.experimental import pallas as pl
from jax.experimental.pallas import tpu as pltpu
from jax.experimental.pallas import tpu_sc as plsc


def _matmul_body(x_ref, w_ref, o_ref):
    o_ref[...] = jnp.dot(x_ref[...], w_ref[...],
                         preferred_element_type=jnp.float32)


def _matmul(x, W, row_block=1000):
    n, d = x.shape
    c = W.shape[1]
    grid = (n // row_block,)
    return pl.pallas_call(
        _matmul_body,
        grid=grid,
        in_specs=[
            pl.BlockSpec((row_block, d), lambda i: (i, 0)),
            pl.BlockSpec((d, c), lambda i: (0, 0)),
        ],
        out_specs=pl.BlockSpec((row_block, c), lambda i: (i, 0)),
        out_shape=jax.ShapeDtypeStruct((n, c), jnp.float32),
    )(x, W)


_BLK = 128


def _make_sc_agg(n_pad, e_pad, n_classes):
    info = plsc.get_sparse_core_info()
    nc, ns = info.num_cores, info.num_subcores
    nw = nc * ns
    epw = e_pad // nw
    n_blk = epw // _BLK
    rows_per_tile = n_pad // ns

    mesh = plsc.VectorSubcoreMesh(core_axis_name="c", subcore_axis_name="s")

    @functools.partial(
        pl.kernel,
        mesh=mesh,
        compiler_params=pltpu.CompilerParams(use_tc_tiling_on_sc=False),
        out_type=jax.ShapeDtypeStruct((nc, n_pad, n_classes), jnp.float32),
        scratch_types=[
            pltpu.VMEM((_BLK,), jnp.int32),
            pltpu.VMEM((_BLK,), jnp.int32),
            pltpu.VMEM((_BLK,), jnp.float32),
            pltpu.VMEM((_BLK, n_classes), jnp.float32),
            pltpu.VMEM((n_pad // ns, n_classes), jnp.float32),
            pltpu.VMEM_SHARED((n_pad, n_classes), jnp.float32),
            pltpu.SemaphoreType.DMA,
        ],
    )
    def sc_agg(src_hbm, dst_hbm, w_hbm, h_hbm, out_hbm,
               sidx, didx, wv, rows, zbuf, acc, gsem):
        cid = lax.axis_index("c")
        sid = lax.axis_index("s")
        wid = sid * nc + cid
        my_out_base = sid * rows_per_tile

        def _zero(i, _):
            zbuf[i, :] = jnp.zeros((n_classes,), jnp.float32)
            return 0
        lax.fori_loop(0, rows_per_tile, _zero, 0)
        pltpu.sync_copy(zbuf, acc.at[pl.ds(my_out_base, rows_per_tile)])
        plsc.subcore_barrier()

        def _block(i, _):
            base = wid * epw + i * _BLK
            pltpu.sync_copy(src_hbm.at[pl.ds(base, _BLK)], sidx)
            pltpu.sync_copy(w_hbm.at[pl.ds(base, _BLK)], wv)
            pltpu.async_copy(h_hbm.at[sidx], rows, gsem).wait()
            for j0 in range(0, _BLK, 16):
                w16 = wv[pl.ds(j0, 16)]
                for j in range(16):
                    rows[j0 + j, :] = rows[j0 + j, :] * jnp.broadcast_to(
                        w16[j], (n_classes,))
            pltpu.sync_copy(dst_hbm.at[pl.ds(base, _BLK)], didx)
            pltpu.sync_copy(rows, acc.at[didx], add=True)
            return 0
        lax.fori_loop(0, n_blk, _block, 0)
        plsc.subcore_barrier()

        pltpu.sync_copy(acc.at[pl.ds(my_out_base, rows_per_tile)],
                        out_hbm.at[cid, pl.ds(my_out_base, rows_per_tile)])

    return sc_agg


def _lsm_body(p_ref, b_ref, o_ref):
    s = p_ref[0] + p_ref[1] + b_ref[...]
    m = jnp.max(s, axis=1, keepdims=True)
    e = jnp.exp(s - m)
    lse = jnp.log(jnp.sum(e, axis=1, keepdims=True))
    o_ref[...] = s - m - lse


def _log_softmax(parts, b, n_out, row_block=1000):
    c = parts.shape[-1]
    grid = (n_out // row_block,)
    return pl.pallas_call(
        _lsm_body,
        grid=grid,
        in_specs=[
            pl.BlockSpec((2, row_block, c), lambda i: (0, i, 0)),
            pl.BlockSpec((c,), lambda i: (0,)),
        ],
        out_specs=pl.BlockSpec((row_block, c), lambda i: (i, 0)),
        out_shape=jax.ShapeDtypeStruct((n_out, c), jnp.float32),
    )(parts, b)


@jax.jit
def kernel(x, edge_index, edge_weight, W, b):
    n_nodes = x.shape[0]
    n_edges = edge_index.shape[1]
    n_classes = W.shape[1]

    info = plsc.get_sparse_core_info()
    nw = info.num_cores * info.num_subcores
    chunk = nw * _BLK
    e_pad = ((n_edges + chunk - 1) // chunk) * chunk
    n_pad = ((n_nodes + nw * 4 - 1) // (nw * 4)) * (nw * 4)

    src = jnp.pad(edge_index[0], (0, e_pad - n_edges))
    dst = jnp.pad(edge_index[1], (0, e_pad - n_edges))
    w = jnp.pad(edge_weight, (0, e_pad - n_edges))

    h = _matmul(x, W)
    parts = _make_sc_agg(n_pad, e_pad, n_classes)(src, dst, w, h)
    return _log_softmax(parts, b, n_out=n_nodes)


# speedup vs baseline: 13.2874x; 1.9232x over previous
import functools

import jax
import jax.numpy as jnp
from jax import lax
from jax.experimental import pallas as pl
from jax.experimental.pallas import tpu as pltpu
from jax.experimental.pallas import tpu_sc as plsc


def _matmul_body(x_ref, w_ref, o_ref):
    o_ref[...] = jnp.dot(x_ref[...], w_ref[...],
                         preferred_element_type=jnp.float32)


def _matmul(x, W, row_block=1000):
    n, d = x.shape
    c = W.shape[1]
    grid = (n // row_block,)
    return pl.pallas_call(
        _matmul_body,
        grid=grid,
        in_specs=[
            pl.BlockSpec((row_block, d), lambda i: (i, 0)),
            pl.BlockSpec((d, c), lambda i: (0, 0)),
        ],
        out_specs=pl.BlockSpec((row_block, c), lambda i: (i, 0)),
        out_shape=jax.ShapeDtypeStruct((n, c), jnp.float32),
    )(x, W)


_BLK = 128


def _make_sc_agg(n_pad, e_pad, n_classes):
    info = plsc.get_sparse_core_info()
    nc, ns = info.num_cores, info.num_subcores
    nw = nc * ns
    epw = e_pad // nw
    n_blk = epw // _BLK
    rows_per_tile = n_pad // ns

    mesh = plsc.VectorSubcoreMesh(core_axis_name="c", subcore_axis_name="s")

    @functools.partial(
        pl.kernel,
        mesh=mesh,
        compiler_params=pltpu.CompilerParams(use_tc_tiling_on_sc=False),
        out_type=jax.ShapeDtypeStruct((nc, n_pad, n_classes), jnp.float32),
        scratch_types=[
            pltpu.VMEM((n_blk, _BLK), jnp.int32),
            pltpu.VMEM((n_blk, _BLK), jnp.int32),
            pltpu.VMEM((n_blk, _BLK), jnp.float32),
            pltpu.VMEM((2, _BLK, n_classes), jnp.float32),
            pltpu.VMEM((2, _BLK, n_classes), jnp.float32),
            pltpu.VMEM((n_pad // ns, n_classes), jnp.float32),
            pltpu.VMEM_SHARED((n_pad, n_classes), jnp.float32),
            pltpu.SemaphoreType.DMA,
            pltpu.SemaphoreType.DMA,
            pltpu.SemaphoreType.DMA,
            pltpu.SemaphoreType.DMA,
        ],
    )
    def sc_agg(src_hbm, dst_hbm, w_hbm, h_hbm, out_hbm,
               sidx, didx, wv, rg, rs, zbuf, acc, g0, g1, s0, s1):
        cid = lax.axis_index("c")
        sid = lax.axis_index("s")
        wid = sid * nc + cid
        my_out_base = sid * rows_per_tile
        gsem = (g0, g1)
        ssem = (s0, s1)

        pltpu.sync_copy(src_hbm.at[pl.ds(wid * n_blk, n_blk)], sidx)
        pltpu.sync_copy(dst_hbm.at[pl.ds(wid * n_blk, n_blk)], didx)
        pltpu.sync_copy(w_hbm.at[pl.ds(wid * n_blk, n_blk)], wv)

        def _zero(i, _):
            zbuf[i, :] = jnp.zeros((n_classes,), jnp.float32)
            return 0
        lax.fori_loop(0, rows_per_tile, _zero, 0)
        pltpu.sync_copy(zbuf, acc.at[pl.ds(my_out_base, rows_per_tile)])
        plsc.subcore_barrier()

        def _ring(b, blk):
            pltpu.make_async_copy(h_hbm.at[sidx.at[blk]], rg.at[b],
                                  gsem[b]).wait()

            @pl.when(blk >= 2)
            def _():
                pltpu.make_async_copy(rs.at[b], acc.at[didx.at[blk]],
                                      ssem[b]).wait()

            for j0 in range(0, _BLK, 16):
                w16 = wv[blk, pl.ds(j0, 16)]
                for j in range(16):
                    rs[b, j0 + j, :] = rg[b, j0 + j, :] * jnp.broadcast_to(
                        w16[j], (n_classes,))

            @pl.when(blk + 2 < n_blk)
            def _():
                pltpu.async_copy(h_hbm.at[sidx.at[blk + 2]], rg.at[b],
                                 gsem[b])
            pltpu.async_copy(rs.at[b], acc.at[didx.at[blk]], ssem[b],
                             add=True)

        pltpu.async_copy(h_hbm.at[sidx.at[0]], rg.at[0], gsem[0])
        pltpu.async_copy(h_hbm.at[sidx.at[1]], rg.at[1], gsem[1])

        def _pair(it, _):
            _ring(0, 2 * it)
            _ring(1, 2 * it + 1)
            return 0
        lax.fori_loop(0, n_blk // 2, _pair, 0)
        pltpu.make_async_copy(rs.at[0], acc.at[didx.at[n_blk - 2]],
                              ssem[0]).wait()
        pltpu.make_async_copy(rs.at[1], acc.at[didx.at[n_blk - 1]],
                              ssem[1]).wait()
        plsc.subcore_barrier()

        pltpu.sync_copy(acc.at[pl.ds(my_out_base, rows_per_tile)],
                        out_hbm.at[cid, pl.ds(my_out_base, rows_per_tile)])

    return sc_agg


def _lsm_body(p_ref, b_ref, o_ref):
    s = p_ref[0] + p_ref[1] + b_ref[...]
    m = jnp.max(s, axis=1, keepdims=True)
    e = jnp.exp(s - m)
    lse = jnp.log(jnp.sum(e, axis=1, keepdims=True))
    o_ref[...] = s - m - lse


def _log_softmax(parts, b, n_out, row_block=1000):
    c = parts.shape[-1]
    grid = (n_out // row_block,)
    return pl.pallas_call(
        _lsm_body,
        grid=grid,
        in_specs=[
            pl.BlockSpec((2, row_block, c), lambda i: (0, i, 0)),
            pl.BlockSpec((c,), lambda i: (0,)),
        ],
        out_specs=pl.BlockSpec((row_block, c), lambda i: (i, 0)),
        out_shape=jax.ShapeDtypeStruct((n_out, c), jnp.float32),
    )(parts, b)


@jax.jit
def kernel(x, edge_index, edge_weight, W, b):
    n_nodes = x.shape[0]
    n_edges = edge_index.shape[1]
    n_classes = W.shape[1]

    info = plsc.get_sparse_core_info()
    nw = info.num_cores * info.num_subcores
    chunk = nw * _BLK * 2
    e_pad = ((n_edges + chunk - 1) // chunk) * chunk
    n_pad = ((n_nodes + nw * 4 - 1) // (nw * 4)) * (nw * 4)

    pad = (0, e_pad - n_edges)
    src = jnp.pad(edge_index[0], pad).reshape(e_pad // _BLK, _BLK)
    dst = jnp.pad(edge_index[1], pad).reshape(e_pad // _BLK, _BLK)
    w = jnp.pad(edge_weight, pad).reshape(e_pad // _BLK, _BLK)

    h = _matmul(x, W)
    parts = _make_sc_agg(n_pad, e_pad, n_classes)(src, dst, w, h)
    return _log_softmax(parts, b, n_out=n_nodes)


# speedup vs baseline: 18.3709x; 1.3826x over previous
import functools

import jax
import jax.numpy as jnp
from jax import lax
from jax.experimental import pallas as pl
from jax.experimental.pallas import tpu as pltpu
from jax.experimental.pallas import tpu_sc as plsc


def _matmul_body(x_ref, w_ref, o_ref):
    o_ref[...] = jnp.dot(x_ref[...], w_ref[...],
                         preferred_element_type=jnp.float32)


def _matmul(x, W, row_block=2000):
    n, d = x.shape
    c = W.shape[1]
    grid = (n // row_block,)
    return pl.pallas_call(
        _matmul_body,
        grid=grid,
        in_specs=[
            pl.BlockSpec((row_block, d), lambda i: (i, 0)),
            pl.BlockSpec((d, c), lambda i: (0, 0)),
        ],
        out_specs=pl.BlockSpec((row_block, c), lambda i: (i, 0)),
        out_shape=jax.ShapeDtypeStruct((n, c), jnp.float32),
    )(x, W)


_BLK = 128


def _make_sc_agg(n_pad, n_edges, n_classes):
    info = plsc.get_sparse_core_info()
    nc, ns = info.num_cores, info.num_subcores
    nw = nc * ns
    epw = n_edges // nw
    n_full = (epw // _BLK) // 2 * 2
    tail = epw - n_full * _BLK
    rows_per_tile = n_pad // ns

    mesh = plsc.VectorSubcoreMesh(core_axis_name="c", subcore_axis_name="s")

    @functools.partial(
        pl.kernel,
        mesh=mesh,
        compiler_params=pltpu.CompilerParams(use_tc_tiling_on_sc=False),
        out_type=jax.ShapeDtypeStruct((nc, n_pad, n_classes), jnp.float32),
        scratch_types=[
            pltpu.VMEM((epw,), jnp.int32),
            pltpu.VMEM((epw,), jnp.int32),
            pltpu.VMEM((epw,), jnp.float32),
            pltpu.VMEM((2, _BLK), jnp.int32),
            pltpu.VMEM((16,), jnp.int32),
            pltpu.VMEM((2, _BLK, n_classes), jnp.float32),
            pltpu.VMEM((2, _BLK, n_classes), jnp.float32),
            pltpu.VMEM((n_pad // ns, n_classes), jnp.float32),
            pltpu.VMEM_SHARED((n_pad, n_classes), jnp.float32),
            pltpu.SemaphoreType.DMA,
            pltpu.SemaphoreType.DMA,
            pltpu.SemaphoreType.DMA,
            pltpu.SemaphoreType.DMA,
        ],
    )
    def sc_agg(ei_hbm, w_hbm, h_hbm, out_hbm,
               sidx, didx, wv, dblk, dtail, rg, rs, zbuf, acc, g0, g1, s0, s1):
        cid = lax.axis_index("c")
        sid = lax.axis_index("s")
        wid = sid * nc + cid
        my_out_base = sid * rows_per_tile
        ebase = wid * epw
        gsem = (g0, g1)
        ssem = (s0, s1)

        pltpu.sync_copy(ei_hbm.at[0, pl.ds(ebase, epw)], sidx)
        pltpu.sync_copy(ei_hbm.at[1, pl.ds(ebase, epw)], didx)
        pltpu.sync_copy(w_hbm.at[pl.ds(ebase, epw)], wv)

        def _zero(i, _):
            for u in range(4):
                zbuf[4 * i + u, :] = jnp.zeros((n_classes,), jnp.float32)
            return 0
        lax.fori_loop(0, rows_per_tile // 4, _zero, 0)
        pltpu.sync_copy(zbuf, acc.at[pl.ds(my_out_base, rows_per_tile)])
        plsc.subcore_barrier()

        def _ring(b, blk):
            pltpu.make_async_copy(h_hbm.at[sidx.at[pl.ds(0, _BLK)]],
                                  rg.at[b], gsem[b]).wait()

            @pl.when(blk >= 2)
            def _():
                pltpu.make_async_copy(rs.at[b], acc.at[dblk.at[b]],
                                      ssem[b]).wait()

            for j0 in range(0, _BLK, 16):
                w16 = wv[pl.ds(blk * _BLK + j0, 16)]
                dblk[b, pl.ds(j0, 16)] = didx[pl.ds(blk * _BLK + j0, 16)]
                for j in range(16):
                    rs[b, j0 + j, :] = rg[b, j0 + j, :] * jnp.broadcast_to(
                        w16[j], (n_classes,))

            @pl.when(blk + 2 < n_full)
            def _():
                pltpu.async_copy(
                    h_hbm.at[sidx.at[pl.ds((blk + 2) * _BLK, _BLK)]],
                    rg.at[b], gsem[b])
            pltpu.async_copy(rs.at[b], acc.at[dblk.at[b]], ssem[b],
                             add=True)

        pltpu.async_copy(h_hbm.at[sidx.at[pl.ds(0, _BLK)]], rg.at[0], gsem[0])
        pltpu.async_copy(h_hbm.at[sidx.at[pl.ds(_BLK, _BLK)]], rg.at[1],
                         gsem[1])

        def _pair(it, _):
            _ring(0, 2 * it)
            _ring(1, 2 * it + 1)
            return 0
        lax.fori_loop(0, n_full // 2, _pair, 0)
        pltpu.make_async_copy(rs.at[0], acc.at[dblk.at[0]], ssem[0]).wait()
        pltpu.make_async_copy(rs.at[1], acc.at[dblk.at[1]], ssem[1]).wait()

        if tail:
            t0 = n_full * _BLK
            pltpu.async_copy(h_hbm.at[sidx.at[pl.ds(t0, tail)]],
                             rg.at[0, pl.ds(0, tail)], gsem[0])
            pltpu.make_async_copy(h_hbm.at[sidx.at[pl.ds(t0, tail)]],
                                  rg.at[0, pl.ds(0, tail)], gsem[0]).wait()
            for j0 in range(0, tail, 16):
                w16 = wv[pl.ds(t0 + j0, 16)]
                dtail[...] = didx[pl.ds(t0 + j0, 16)]
                for j in range(16):
                    rs[0, j0 + j, :] = rg[0, j0 + j, :] * jnp.broadcast_to(
                        w16[j], (n_classes,))
                pltpu.sync_copy(rs.at[0, pl.ds(j0, 16)], acc.at[dtail],
                                add=True)

        plsc.subcore_barrier()

        pltpu.sync_copy(acc.at[pl.ds(my_out_base, rows_per_tile)],
                        out_hbm.at[cid, pl.ds(my_out_base, rows_per_tile)])

    return sc_agg


def _lsm_body(p_ref, b_ref, o_ref):
    s = p_ref[0] + p_ref[1] + b_ref[...]
    m = jnp.max(s, axis=1, keepdims=True)
    e = jnp.exp(s - m)
    lse = jnp.log(jnp.sum(e, axis=1, keepdims=True))
    o_ref[...] = s - m - lse


def _log_softmax(parts, b, n_out, row_block=2000):
    c = parts.shape[-1]
    grid = (n_out // row_block,)
    return pl.pallas_call(
        _lsm_body,
        grid=grid,
        in_specs=[
            pl.BlockSpec((2, row_block, c), lambda i: (0, i, 0)),
            pl.BlockSpec((c,), lambda i: (0,)),
        ],
        out_specs=pl.BlockSpec((row_block, c), lambda i: (i, 0)),
        out_shape=jax.ShapeDtypeStruct((n_out, c), jnp.float32),
    )(parts, b)


@jax.jit
def kernel(x, edge_index, edge_weight, W, b):
    n_nodes = x.shape[0]
    n_edges = edge_index.shape[1]
    n_classes = W.shape[1]

    info = plsc.get_sparse_core_info()
    nw = info.num_cores * info.num_subcores
    if n_edges % (nw * 16):
        e_pad = ((n_edges + nw * 16 - 1) // (nw * 16)) * (nw * 16)
        edge_index = jnp.pad(edge_index, ((0, 0), (0, e_pad - n_edges)))
        edge_weight = jnp.pad(edge_weight, (0, e_pad - n_edges))
        n_edges = e_pad
    n_pad = ((n_nodes + nw * 4 - 1) // (nw * 4)) * (nw * 4)

    h = _matmul(x, W)
    parts = _make_sc_agg(n_pad, n_edges, n_classes)(edge_index, edge_weight, h)
    return _log_softmax(parts, b, n_out=n_nodes)


# speedup vs baseline: 23.6619x; 1.2880x over previous
import functools

import jax
import jax.numpy as jnp
from jax import lax
from jax.experimental import pallas as pl
from jax.experimental.pallas import tpu as pltpu
from jax.experimental.pallas import tpu_sc as plsc


def _matmul_body(x_ref, w_ref, o_ref):
    x = x_ref[...]
    w = w_ref[...]
    n, d = x.shape
    xr = x.reshape(n // 8, 8, d)
    hs = [jnp.dot(xr[:, u, :], w, preferred_element_type=jnp.float32)
          for u in range(8)]
    o_ref[...] = jnp.concatenate(hs, axis=1)


def _matmul(x, W):
    n, d = x.shape
    c = W.shape[1]
    return pl.pallas_call(
        _matmul_body,
        out_shape=jax.ShapeDtypeStruct((n * c // 128, 128), jnp.float32),
    )(x, W)


_BLK = 128


def _make_sc_agg(n_pad, n_edges, n_classes):
    info = plsc.get_sparse_core_info()
    nc, ns = info.num_cores, info.num_subcores
    nw = nc * ns
    epw = n_edges // nw
    n_full = (epw // _BLK) // 6 * 6
    tail = epw - n_full * _BLK
    rows_per_tile = n_pad // ns

    mesh = plsc.VectorSubcoreMesh(core_axis_name="c", subcore_axis_name="s")

    @functools.partial(
        pl.kernel,
        mesh=mesh,
        compiler_params=pltpu.CompilerParams(use_tc_tiling_on_sc=False),
        out_type=jax.ShapeDtypeStruct((nc, n_pad, n_classes), jnp.float32),
        scratch_types=[
            pltpu.VMEM((epw,), jnp.int32),
            pltpu.VMEM((epw,), jnp.int32),
            pltpu.VMEM((epw,), jnp.float32),
            pltpu.VMEM((2, _BLK), jnp.int32),
            pltpu.VMEM((16,), jnp.int32),
            pltpu.VMEM((3, _BLK, n_classes), jnp.float32),
            pltpu.VMEM((2, _BLK, n_classes), jnp.float32),
            pltpu.VMEM((n_pad // ns, n_classes), jnp.float32),
            pltpu.VMEM_SHARED((n_pad, n_classes), jnp.float32),
            pltpu.SemaphoreType.DMA,
            pltpu.SemaphoreType.DMA,
            pltpu.SemaphoreType.DMA,
            pltpu.SemaphoreType.DMA,
            pltpu.SemaphoreType.DMA,
        ],
    )
    def sc_agg(ei_hbm, w_hbm, h_hbm, out_hbm, sidx, didx, wv, dblk, dtail,
               rg, rs, zbuf, acc, g0, g1, g2, s0, s1):
        cid = lax.axis_index("c")
        sid = lax.axis_index("s")
        wid = sid * nc + cid
        my_out_base = sid * rows_per_tile
        ebase = wid * epw
        gsem = (g0, g1, g2)
        ssem = (s0, s1)

        pltpu.sync_copy(ei_hbm.at[0, pl.ds(ebase, epw)], sidx)
        pltpu.sync_copy(ei_hbm.at[1, pl.ds(ebase, epw)], didx)
        pltpu.sync_copy(w_hbm.at[pl.ds(ebase, epw)], wv)

        def _zero(i, _):
            for u in range(4):
                zbuf[4 * i + u, :] = jnp.zeros((n_classes,), jnp.float32)
            return 0
        lax.fori_loop(0, rows_per_tile // 4, _zero, 0)
        pltpu.sync_copy(zbuf, acc.at[pl.ds(my_out_base, rows_per_tile)])
        plsc.subcore_barrier()

        def _gather(gb, blk):
            pltpu.async_copy(h_hbm.at[sidx.at[pl.ds(blk * _BLK, _BLK)]],
                             rg.at[gb], gsem[gb])

        def _ring(gb, sb, blk):
            pltpu.make_async_copy(h_hbm.at[sidx.at[pl.ds(0, _BLK)]],
                                  rg.at[gb], gsem[gb]).wait()

            @pl.when(blk >= 2)
            def _():
                pltpu.make_async_copy(rs.at[sb], acc.at[dblk.at[sb]],
                                      ssem[sb]).wait()

            for j0 in range(0, _BLK, 16):
                w16 = wv[pl.ds(blk * _BLK + j0, 16)]
                dblk[sb, pl.ds(j0, 16)] = didx[pl.ds(blk * _BLK + j0, 16)]
                for j in range(16):
                    rs[sb, j0 + j, :] = rg[gb, j0 + j, :] * jnp.broadcast_to(
                        w16[j], (n_classes,))

            @pl.when(blk + 3 < n_full)
            def _():
                _gather(gb, blk + 3)
            pltpu.async_copy(rs.at[sb], acc.at[dblk.at[sb]], ssem[sb],
                             add=True)

        _gather(0, 0)
        _gather(1, 1)
        _gather(2, 2)

        def _six(it, _):
            for u in range(6):
                _ring(u % 3, u % 2, 6 * it + u)
            return 0
        lax.fori_loop(0, n_full // 6, _six, 0)
        pltpu.make_async_copy(rs.at[0], acc.at[dblk.at[0]], ssem[0]).wait()
        pltpu.make_async_copy(rs.at[1], acc.at[dblk.at[1]], ssem[1]).wait()

        if tail:
            t0 = n_full * _BLK
            for j0 in range(0, tail, _BLK):
                seg = min(_BLK, tail - j0)
                pltpu.async_copy(h_hbm.at[sidx.at[pl.ds(t0 + j0, seg)]],
                                 rg.at[0, pl.ds(0, seg)], gsem[0])
                pltpu.make_async_copy(h_hbm.at[sidx.at[pl.ds(t0 + j0, seg)]],
                                      rg.at[0, pl.ds(0, seg)], gsem[0]).wait()
                for k0 in range(0, seg, 16):
                    w16 = wv[pl.ds(t0 + j0 + k0, 16)]
                    dtail[...] = didx[pl.ds(t0 + j0 + k0, 16)]
                    for j in range(16):
                        rs[0, k0 + j, :] = (rg[0, k0 + j, :] *
                                            jnp.broadcast_to(w16[j],
                                                             (n_classes,)))
                    pltpu.sync_copy(rs.at[0, pl.ds(k0, 16)], acc.at[dtail],
                                    add=True)

        plsc.subcore_barrier()

        pltpu.sync_copy(acc.at[pl.ds(my_out_base, rows_per_tile)],
                        out_hbm.at[cid, pl.ds(my_out_base, rows_per_tile)])

    return sc_agg


def _lsm_body(p_ref, b_ref, o_ref):
    p = p_ref[...]
    n_pk = o_ref.shape[0]
    s128 = (p[0] + p[1])[:n_pk]
    b = b_ref[...]
    outs = []
    for u in range(8):
        s = lax.slice(s128, (0, 16 * u), (n_pk, 16 * u + 16)) + b
        m = jnp.max(s, axis=1, keepdims=True)
        e = jnp.exp(s - m)
        lse = jnp.log(jnp.sum(e, axis=1, keepdims=True))
        outs.append(s - m - lse)
    o_ref[...] = jnp.concatenate(outs, axis=1)


def _log_softmax(parts128, b, n_out):
    c = b.shape[0]
    return pl.pallas_call(
        _lsm_body,
        out_shape=jax.ShapeDtypeStruct((n_out * c // 128, 128), jnp.float32),
    )(parts128, b)


@jax.jit
def kernel(x, edge_index, edge_weight, W, b):
    n_nodes = x.shape[0]
    n_edges = edge_index.shape[1]
    n_classes = W.shape[1]

    info = plsc.get_sparse_core_info()
    nw = info.num_cores * info.num_subcores
    if n_edges % (nw * 16):
        e_pad = ((n_edges + nw * 16 - 1) // (nw * 16)) * (nw * 16)
        edge_index = jnp.pad(edge_index, ((0, 0), (0, e_pad - n_edges)))
        edge_weight = jnp.pad(edge_weight, (0, e_pad - n_edges))
        n_edges = e_pad
    n_pad = ((n_nodes + nw * 4 - 1) // (nw * 4)) * (nw * 4)

    h_packed = _matmul(x, W)
    h = h_packed.reshape(n_nodes, n_classes)
    parts = _make_sc_agg(n_pad, n_edges, n_classes)(edge_index, edge_weight, h)
    parts128 = parts.reshape(2, n_pad * n_classes // 128, 128)
    out128 = _log_softmax(parts128, b, n_out=n_nodes)
    return out128.reshape(n_nodes, n_classes)


# speedup vs baseline: 28.5314x; 1.2058x over previous
import functools

import jax
import jax.numpy as jnp
from jax import lax
from jax.experimental import pallas as pl
from jax.experimental.pallas import tpu as pltpu
from jax.experimental.pallas import tpu_sc as plsc


def _matmul_body(x_ref, w_ref, o_ref):
    x = x_ref[...]
    w = w_ref[...]
    n, d = x.shape
    xr = x.reshape(n // 8, 8, d)
    hs = [jnp.dot(xr[:, u, :], w, preferred_element_type=jnp.float32)
          for u in range(8)]
    o_ref[...] = jnp.concatenate(hs, axis=1)


def _matmul(x, W):
    n, d = x.shape
    c = W.shape[1]
    return pl.pallas_call(
        _matmul_body,
        out_shape=jax.ShapeDtypeStruct((n * c // 128, 128), jnp.float32),
    )(x, W)


_BLK = 128


def _make_sc_agg(n_pad, n_nodes, n_edges, n_classes):
    info = plsc.get_sparse_core_info()
    nc, ns = info.num_cores, info.num_subcores
    nw = nc * ns
    epw = n_edges // nw
    n_full = (epw // _BLK) // 3 * 3
    tail = epw - n_full * _BLK
    rows_per_tile = n_pad // ns

    mesh = plsc.VectorSubcoreMesh(core_axis_name="c", subcore_axis_name="s")

    @functools.partial(
        pl.kernel,
        mesh=mesh,
        compiler_params=pltpu.CompilerParams(use_tc_tiling_on_sc=False),
        out_type=jax.ShapeDtypeStruct((nc, n_pad, n_classes), jnp.float32),
        scratch_types=[
            pltpu.VMEM((epw,), jnp.int32),
            pltpu.VMEM((epw,), jnp.int32),
            pltpu.VMEM((epw,), jnp.float32),
            pltpu.VMEM((3, _BLK), jnp.int32),
            pltpu.VMEM((16,), jnp.int32),
            pltpu.VMEM((3, _BLK, n_classes), jnp.float32),
            pltpu.VMEM((3, _BLK, n_classes), jnp.float32),
            pltpu.VMEM((n_pad // ns, n_classes), jnp.float32),
            pltpu.VMEM_SHARED((n_pad, n_classes), jnp.float32),
            pltpu.VMEM_SHARED((n_nodes, n_classes), jnp.float32),
            pltpu.SemaphoreType.DMA,
            pltpu.SemaphoreType.DMA,
            pltpu.SemaphoreType.DMA,
            pltpu.SemaphoreType.DMA,
            pltpu.SemaphoreType.DMA,
            pltpu.SemaphoreType.DMA,
            pltpu.SemaphoreType.DMA,
        ],
    )
    def sc_agg(ei_hbm, w_hbm, h_hbm, out_hbm, sidx, didx, wv, dblk, dtail,
               rg, rs, zbuf, acc, h_sh, g0, g1, g2, s0, s1, s2, hsem):
        cid = lax.axis_index("c")
        sid = lax.axis_index("s")
        wid = sid * nc + cid
        my_out_base = sid * rows_per_tile
        ebase = wid * epw
        gsem = (g0, g1, g2)
        ssem = (s0, s1, s2)

        h_rows = n_nodes // ns
        pltpu.async_copy(h_hbm.at[pl.ds(sid * h_rows, h_rows)],
                         h_sh.at[pl.ds(sid * h_rows, h_rows)], hsem)

        pltpu.sync_copy(ei_hbm.at[0, pl.ds(ebase, epw)], sidx)
        pltpu.sync_copy(ei_hbm.at[1, pl.ds(ebase, epw)], didx)
        pltpu.sync_copy(w_hbm.at[pl.ds(ebase, epw)], wv)

        def _zero(i, _):
            for u in range(4):
                zbuf[4 * i + u, :] = jnp.zeros((n_classes,), jnp.float32)
            return 0
        lax.fori_loop(0, rows_per_tile // 4, _zero, 0)
        pltpu.sync_copy(zbuf, acc.at[pl.ds(my_out_base, rows_per_tile)])
        pltpu.make_async_copy(h_hbm.at[pl.ds(sid * h_rows, h_rows)],
                              h_sh.at[pl.ds(sid * h_rows, h_rows)],
                              hsem).wait()
        plsc.subcore_barrier()

        def _gather(gb, blk):
            pltpu.async_copy(h_sh.at[sidx.at[pl.ds(blk * _BLK, _BLK)]],
                             rg.at[gb], gsem[gb])

        def _ring(gb, sb, blk):
            pltpu.make_async_copy(h_sh.at[sidx.at[pl.ds(0, _BLK)]],
                                  rg.at[gb], gsem[gb]).wait()

            @pl.when(blk >= 3)
            def _():
                pltpu.make_async_copy(rs.at[sb], acc.at[dblk.at[sb]],
                                      ssem[sb]).wait()

            for j0 in range(0, _BLK, 16):
                w16 = wv[pl.ds(blk * _BLK + j0, 16)]
                dblk[sb, pl.ds(j0, 16)] = didx[pl.ds(blk * _BLK + j0, 16)]
                for j in range(16):
                    rs[sb, j0 + j, :] = rg[gb, j0 + j, :] * jnp.broadcast_to(
                        w16[j], (n_classes,))

            @pl.when(blk + 3 < n_full)
            def _():
                _gather(gb, blk + 3)
            pltpu.async_copy(rs.at[sb], acc.at[dblk.at[sb]], ssem[sb],
                             add=True)

        _gather(0, 0)
        _gather(1, 1)
        _gather(2, 2)

        def _three(it, _):
            for u in range(3):
                _ring(u, u, 3 * it + u)
            return 0
        lax.fori_loop(0, n_full // 3, _three, 0)
        pltpu.make_async_copy(rs.at[0], acc.at[dblk.at[0]], ssem[0]).wait()
        pltpu.make_async_copy(rs.at[1], acc.at[dblk.at[1]], ssem[1]).wait()
        pltpu.make_async_copy(rs.at[2], acc.at[dblk.at[2]], ssem[2]).wait()

        if tail:
            t0 = n_full * _BLK
            for j0 in range(0, tail, _BLK):
                seg = min(_BLK, tail - j0)
                pltpu.async_copy(h_sh.at[sidx.at[pl.ds(t0 + j0, seg)]],
                                 rg.at[0, pl.ds(0, seg)], gsem[0])
                pltpu.make_async_copy(h_sh.at[sidx.at[pl.ds(t0 + j0, seg)]],
                                      rg.at[0, pl.ds(0, seg)], gsem[0]).wait()
                for k0 in range(0, seg, 16):
                    w16 = wv[pl.ds(t0 + j0 + k0, 16)]
                    dtail[...] = didx[pl.ds(t0 + j0 + k0, 16)]
                    for j in range(16):
                        rs[0, k0 + j, :] = (rg[0, k0 + j, :] *
                                            jnp.broadcast_to(w16[j],
                                                             (n_classes,)))
                    pltpu.sync_copy(rs.at[0, pl.ds(k0, 16)], acc.at[dtail],
                                    add=True)

        plsc.subcore_barrier()

        pltpu.sync_copy(acc.at[pl.ds(my_out_base, rows_per_tile)],
                        out_hbm.at[cid, pl.ds(my_out_base, rows_per_tile)])

    return sc_agg


def _lsm_body(p_ref, b_ref, o_ref):
    p = p_ref[...]
    n_pk = o_ref.shape[0]
    s128 = (p[0] + p[1])[:n_pk]
    b = b_ref[...]
    outs = []
    for u in range(8):
        s = lax.slice(s128, (0, 16 * u), (n_pk, 16 * u + 16)) + b
        m = jnp.max(s, axis=1, keepdims=True)
        e = jnp.exp(s - m)
        lse = jnp.log(jnp.sum(e, axis=1, keepdims=True))
        outs.append(s - m - lse)
    o_ref[...] = jnp.concatenate(outs, axis=1)


def _log_softmax(parts128, b, n_out):
    c = b.shape[0]
    return pl.pallas_call(
        _lsm_body,
        out_shape=jax.ShapeDtypeStruct((n_out * c // 128, 128), jnp.float32),
    )(parts128, b)


@jax.jit
def kernel(x, edge_index, edge_weight, W, b):
    n_nodes = x.shape[0]
    n_edges = edge_index.shape[1]
    n_classes = W.shape[1]

    info = plsc.get_sparse_core_info()
    nw = info.num_cores * info.num_subcores
    if n_edges % (nw * 16):
        e_pad = ((n_edges + nw * 16 - 1) // (nw * 16)) * (nw * 16)
        edge_index = jnp.pad(edge_index, ((0, 0), (0, e_pad - n_edges)))
        edge_weight = jnp.pad(edge_weight, (0, e_pad - n_edges))
        n_edges = e_pad
    n_pad = ((n_nodes + nw * 4 - 1) // (nw * 4)) * (nw * 4)

    h_packed = _matmul(x, W)
    h = h_packed.reshape(n_nodes, n_classes)
    parts = _make_sc_agg(n_pad, n_nodes, n_edges, n_classes)(
        edge_index, edge_weight, h)
    parts128 = parts.reshape(2, n_pad * n_classes // 128, 128)
    out128 = _log_softmax(parts128, b, n_out=n_nodes)
    return out128.reshape(n_nodes, n_classes)
